# Initial kernel scaffold; baseline (speedup 1.0000x reference)
import jax
import jax.numpy as jnp
from jax.experimental import pallas as pl


def kernel(comm_obs, my_id, pos_table, token_table, W, b):
    raise NotImplementedError("write your pallas kernel here")


# speedup vs baseline: 10.6093x; 10.6093x over previous
import functools

import jax
import jax.numpy as jnp
from jax import lax
from jax.experimental import pallas as pl
from jax.experimental.pallas import tpu as pltpu
from jax.experimental.pallas import tpu_sc as plsc

_MAP = 8000
_DS = 8
_GRID = _MAP // _DS
_L = 32
_B = 16384
_E = 16
_OUT = 128

_NCORES = 2
_NSUB = 16
_NW = _NCORES * _NSUB
_BPW = _B // _NW
_NC = 64
_NCHUNK = _BPW // _NC
_GCHUNK = 128


def _sc_gather(obs_flat, my_id, pos_table, token_table):
  mesh = plsc.VectorSubcoreMesh(core_axis_name="c", subcore_axis_name="s")

  @functools.partial(
      pl.kernel,
      out_type=(
          jax.ShapeDtypeStruct((_B, _E), jnp.float32),
          jax.ShapeDtypeStruct((_B * _L, _E), jnp.float32),
          jax.ShapeDtypeStruct((_B * _L, _E), jnp.float32),
      ),
      mesh=mesh,
      compiler_params=pltpu.CompilerParams(
          needs_layout_passes=False, use_tc_tiling_on_sc=False),
      scratch_types=[
          pltpu.VMEM((_NC * _L * 4,), jnp.int32),
          pltpu.VMEM((_NC,), jnp.int32),
          pltpu.VMEM((_NC * _L,), jnp.int32),
          pltpu.VMEM((_NC * _L,), jnp.int32),
          pltpu.VMEM((_NC,), jnp.int32),
          pltpu.VMEM((_NC * _L, _E), jnp.float32),
          pltpu.VMEM((_NC * _L, _E), jnp.float32),
          pltpu.VMEM((_NC, _E), jnp.float32),
          pltpu.SemaphoreType.DMA,
      ],
  )
  def k(obs_hbm, myid_hbm, pos_hbm, tok_hbm, my_out, pos_out, tok_out,
        obs_v, myid_v, pidx_v, tidx_v, midx_v, posr_v, tokr_v, myr_v, sem):
    wid = lax.axis_index("s") * _NCORES + lax.axis_index("c")

    def chunk(ci, carry):
      base = wid * _BPW + ci * _NC
      pltpu.sync_copy(obs_hbm.at[pl.ds(base * _L * 4, _NC * _L * 4)], obs_v)
      pltpu.sync_copy(myid_hbm.at[pl.ds(base, _NC)], myid_v)

      def grp(g, c2):
        e = g * 16
        flat4 = (e + lax.broadcasted_iota(jnp.int32, (16,), 0)) * 4
        r = plsc.load_gather(obs_v, [flat4 + 1])
        c = plsc.load_gather(obs_v, [flat4 + 2])
        t = plsc.load_gather(obs_v, [flat4 + 3])
        pidx_v[pl.ds(e, 16)] = (r // _DS) * _GRID + (c // _DS)
        tidx_v[pl.ds(e, 16)] = t
        return c2

      lax.fori_loop(0, _NC * _L // 16, grp, 0)

      def bgrp(gb, c2):
        b0 = gb * 16
        row = (b0 + lax.broadcasted_iota(jnp.int32, (16,), 0)) * _L
        myv = myid_v[pl.ds(b0, 16)]
        sel = jnp.zeros((16,), jnp.int32)
        found = jnp.zeros((16,), jnp.bool_)
        for l in range(_L):
          ag = plsc.load_gather(obs_v, [(row + l) * 4])
          m = (ag == myv) & (ag != 0)
          sel = jnp.where(m & (~found), jnp.full((16,), l, jnp.int32), sel)
          found = found | m
        midx_v[pl.ds(b0, 16)] = plsc.load_gather(pidx_v, [row + sel])
        return c2

      lax.fori_loop(0, _NC // 16, bgrp, 0)

      copies = []
      for k0 in range(0, _NC * _L, _GCHUNK):
        copies.append(pltpu.async_copy(
            pos_hbm.at[pidx_v.at[pl.ds(k0, _GCHUNK)]],
            posr_v.at[pl.ds(k0, _GCHUNK)], sem))
        copies.append(pltpu.async_copy(
            tok_hbm.at[tidx_v.at[pl.ds(k0, _GCHUNK)]],
            tokr_v.at[pl.ds(k0, _GCHUNK)], sem))
      copies.append(pltpu.async_copy(pos_hbm.at[midx_v], myr_v, sem))
      for cp in copies:
        cp.wait()

      pltpu.sync_copy(myr_v, my_out.at[pl.ds(base, _NC)])
      pltpu.sync_copy(posr_v, pos_out.at[pl.ds(base * _L, _NC * _L)])
      pltpu.sync_copy(tokr_v, tok_out.at[pl.ds(base * _L, _NC * _L)])
      return carry

    lax.fori_loop(0, _NCHUNK, chunk, 0)

  return k(obs_flat, my_id, pos_table, token_table)


def _tc_linear(myr, posr, tokr, wmy, wp, wt, bias):
  bm = 1024

  def body(my_ref, p_ref, t_ref, wmy_ref, wp_ref, wt_ref, b_ref, o_ref):
    acc = jnp.dot(p_ref[...], wp_ref[...], preferred_element_type=jnp.float32)
    acc = acc + jnp.dot(t_ref[...], wt_ref[...],
                        preferred_element_type=jnp.float32)
    acc = acc + jnp.dot(my_ref[...], wmy_ref[...],
                        preferred_element_type=jnp.float32)
    o_ref[...] = acc + b_ref[...]

  return pl.pallas_call(
      body,
      grid=(_B // bm,),
      in_specs=[
          pl.BlockSpec((bm, _E), lambda i: (i, 0)),
          pl.BlockSpec((bm, _L * _E), lambda i: (i, 0)),
          pl.BlockSpec((bm, _L * _E), lambda i: (i, 0)),
          pl.BlockSpec((_E, _OUT), lambda i: (0, 0)),
          pl.BlockSpec((_L * _E, _OUT), lambda i: (0, 0)),
          pl.BlockSpec((_L * _E, _OUT), lambda i: (0, 0)),
          pl.BlockSpec((1, _OUT), lambda i: (0, 0)),
      ],
      out_specs=pl.BlockSpec((bm, _OUT), lambda i: (i, 0)),
      out_shape=jax.ShapeDtypeStruct((_B, _OUT), jnp.float32),
  )(myr, posr, tokr, wmy, wp, wt, bias)


def kernel(comm_obs, my_id, pos_table, token_table, W, b):
  obs_flat = comm_obs.astype(jnp.int32).reshape(-1)
  myr, posr, tokr = _sc_gather(obs_flat, my_id.astype(jnp.int32),
                               pos_table, token_table)
  posr = posr.reshape(_B, _L * _E)
  tokr = tokr.reshape(_B, _L * _E)
  wmy = W[:, :_E].T
  wr = W[:, _E:].reshape(_OUT, _L, 2, _E)
  wp = wr[:, :, 0, :].reshape(_OUT, _L * _E).T
  wt = wr[:, :, 1, :].reshape(_OUT, _L * _E).T
  return _tc_linear(myr, posr, tokr, wmy, wp, wt, b.reshape(1, _OUT))


# speedup vs baseline: 11.2035x; 1.0560x over previous
import functools

import jax
import jax.numpy as jnp
from jax import lax
from jax.experimental import pallas as pl
from jax.experimental.pallas import tpu as pltpu
from jax.experimental.pallas import tpu_sc as plsc

_MAP = 8000
_DS = 8
_GRID = _MAP // _DS
_L = 32
_B = 16384
_E = 16
_OUT = 128

_NCORES = 2
_NSUB = 16
_NW = _NCORES * _NSUB
_BPW = _B // _NW
_NC = 64
_NCHUNK = _BPW // _NC
_GCHUNK = 128
_NP = _L * _E // 128


_TOKP = 100008


def _sc_gather(obs_flat, my_id, pos_wide, token_wide):
  mesh = plsc.VectorSubcoreMesh(core_axis_name="c", subcore_axis_name="s")

  @functools.partial(
      pl.kernel,
      out_type=(
          jax.ShapeDtypeStruct((_B, _E), jnp.float32),
          jax.ShapeDtypeStruct((_NP, _B * 8, _E), jnp.float32),
          jax.ShapeDtypeStruct((_NP, _B * 8, _E), jnp.float32),
      ),
      mesh=mesh,
      compiler_params=pltpu.CompilerParams(
          needs_layout_passes=False, use_tc_tiling_on_sc=False),
      scratch_types=[
          pltpu.VMEM((_NC * _L * 4,), jnp.int32),
          pltpu.VMEM((_NC,), jnp.int32),
          pltpu.VMEM((_NC * _L,), jnp.int32),
          pltpu.VMEM((_NC * _L,), jnp.int32),
          pltpu.VMEM((_NC,), jnp.int32),
          pltpu.VMEM((_NC * _L, _E), jnp.float32),
          pltpu.VMEM((_NC * _L, _E), jnp.float32),
          pltpu.VMEM((_NC, _E), jnp.float32),
          pltpu.SemaphoreType.DMA,
      ],
  )
  def k(obs_hbm, myid_hbm, pos_tab, tok_tab, my_out, pos_out, tok_out,
        obs_v, myid_v, pidx_v, tidx_v, midx_v, posw_v, tokw_v, myr_v, sem):
    wid = lax.axis_index("s") * _NCORES + lax.axis_index("c")

    def chunk(ci, carry):
      base = wid * _BPW + ci * _NC
      pltpu.sync_copy(obs_hbm.at[pl.ds(base * _L * 4, _NC * _L * 4)], obs_v)
      pltpu.sync_copy(myid_hbm.at[pl.ds(base, _NC)], myid_v)

      def grp(g, c2):
        e = g * 16
        d = e + lax.broadcasted_iota(jnp.int32, (16,), 0)
        j = d // (_NC * 8)
        rem = d % (_NC * 8)
        b = rem // 8
        t = rem % 8
        src4 = (b * _L + j * 8 + t) * 4
        r = plsc.load_gather(obs_v, [src4 + 1])
        c = plsc.load_gather(obs_v, [src4 + 2])
        tk = plsc.load_gather(obs_v, [src4 + 3])
        pidx_v[pl.ds(e, 16)] = (r // _DS) * _GRID + (c // _DS)
        tidx_v[pl.ds(e, 16)] = tk
        return c2

      lax.fori_loop(0, _NC * _L // 16, grp, 0)

      def bgrp(gb, c2):
        b0 = gb * 16
        bloc = b0 + lax.broadcasted_iota(jnp.int32, (16,), 0)
        row4 = bloc * (_L * 4)
        myv = myid_v[pl.ds(b0, 16)]
        sel = jnp.zeros((16,), jnp.int32)
        found = jnp.zeros((16,), jnp.bool_)
        for l in range(_L):
          ag = plsc.load_gather(obs_v, [row4 + l * 4])
          m = (ag == myv) & (ag != 0)
          sel = jnp.where(m & (~found), jnp.full((16,), l, jnp.int32), sel)
          found = found | m
        dsel = (sel // 8) * (_NC * 8) + bloc * 8 + sel % 8
        midx_v[pl.ds(b0, 16)] = plsc.load_gather(pidx_v, [dsel])
        return c2

      lax.fori_loop(0, _NC // 16, bgrp, 0)

      copies = []
      for k0 in range(0, _NC * _L, _GCHUNK):
        copies.append(pltpu.async_copy(
            pos_tab.at[pidx_v.at[pl.ds(k0, _GCHUNK)]],
            posw_v.at[pl.ds(k0, _GCHUNK)], sem))
        copies.append(pltpu.async_copy(
            tok_tab.at[tidx_v.at[pl.ds(k0, _GCHUNK)]],
            tokw_v.at[pl.ds(k0, _GCHUNK)], sem))
      copies.append(pltpu.async_copy(pos_tab.at[midx_v], myr_v, sem))
      for cp in copies:
        cp.wait()

      for j in range(_NP):
        pltpu.sync_copy(posw_v.at[pl.ds(j * _NC * 8, _NC * 8)],
                        pos_out.at[j, pl.ds(base * 8, _NC * 8)])
        pltpu.sync_copy(tokw_v.at[pl.ds(j * _NC * 8, _NC * 8)],
                        tok_out.at[j, pl.ds(base * 8, _NC * 8)])
      pltpu.sync_copy(myr_v, my_out.at[pl.ds(base, _NC)])
      return carry

    lax.fori_loop(0, _NCHUNK, chunk, 0)

  return k(obs_flat, my_id, pos_wide, token_wide)


def _tc_linear(myr, posr, tokr, wmy, wp, wt, bias):
  bm = 1024

  def body(my_ref, p_ref, t_ref, wmy_ref, wp_ref, wt_ref, b_ref, o_ref):
    acc = jnp.dot(my_ref[...], wmy_ref[...],
                  preferred_element_type=jnp.float32)
    for j in range(_NP):
      acc = acc + jnp.dot(p_ref[j], wp_ref[j],
                          preferred_element_type=jnp.float32)
      acc = acc + jnp.dot(t_ref[j], wt_ref[j],
                          preferred_element_type=jnp.float32)
    o_ref[...] = acc + b_ref[...]

  return pl.pallas_call(
      body,
      grid=(_B // bm,),
      in_specs=[
          pl.BlockSpec((bm, _E), lambda i: (i, 0)),
          pl.BlockSpec((_NP, bm, 128), lambda i: (0, i, 0)),
          pl.BlockSpec((_NP, bm, 128), lambda i: (0, i, 0)),
          pl.BlockSpec((_E, _OUT), lambda i: (0, 0)),
          pl.BlockSpec((_NP, 128, _OUT), lambda i: (0, 0, 0)),
          pl.BlockSpec((_NP, 128, _OUT), lambda i: (0, 0, 0)),
          pl.BlockSpec((1, _OUT), lambda i: (0, 0)),
      ],
      out_specs=pl.BlockSpec((bm, _OUT), lambda i: (i, 0)),
      out_shape=jax.ShapeDtypeStruct((_B, _OUT), jnp.float32),
  )(myr, posr, tokr, wmy, wp, wt, bias)


def kernel(comm_obs, my_id, pos_table, token_table, W, b):
  obs_flat = comm_obs.astype(jnp.int32).reshape(-1)
  myr, posr, tokr = _sc_gather(obs_flat, my_id.astype(jnp.int32),
                               pos_table, token_table)
  posr = posr.reshape(_NP, _B, 128)
  tokr = tokr.reshape(_NP, _B, 128)
  wmy = W[:, :_E].T
  wr = W[:, _E:].reshape(_OUT, _L, 2, _E)
  wp = wr[:, :, 0, :].reshape(_OUT, _NP, 8 * _E).transpose(1, 2, 0)
  wt = wr[:, :, 1, :].reshape(_OUT, _NP, 8 * _E).transpose(1, 2, 0)
  return _tc_linear(myr, posr, tokr, wmy, wp, wt, b.reshape(1, _OUT))


# speedup vs baseline: 22.7225x; 2.0282x over previous
import functools

import jax
import jax.numpy as jnp
from jax import lax
from jax.experimental import pallas as pl
from jax.experimental.pallas import tpu as pltpu
from jax.experimental.pallas import tpu_sc as plsc

_MAP = 8000
_DS = 8
_GRID = _MAP // _DS
_L = 32
_B = 16384
_E = 16
_OUT = 128

_NCORES = 2
_NSUB = 16
_NW = _NCORES * _NSUB
_BPW = _B // _NW
_NC = 64
_NCHUNK = _BPW // _NC
_GCHUNK = 128
_NP = _L * _E // 128


_TOKP = 100008


def _sc_gather(obs4, my_id, pos_wide, token_wide):
  mesh = plsc.VectorSubcoreMesh(core_axis_name="c", subcore_axis_name="s")

  @functools.partial(
      pl.kernel,
      out_type=(
          jax.ShapeDtypeStruct((_B, _E), jnp.float32),
          jax.ShapeDtypeStruct((_NP, _B * 8, _E), jnp.float32),
          jax.ShapeDtypeStruct((_NP, _B * 8, _E), jnp.float32),
      ),
      mesh=mesh,
      compiler_params=pltpu.CompilerParams(
          needs_layout_passes=False, use_tc_tiling_on_sc=False),
      scratch_types=[
          pltpu.VMEM((_L, 4, _NC), jnp.int32),
          pltpu.VMEM((_NC,), jnp.int32),
          pltpu.VMEM((_NC * _L,), jnp.int32),
          pltpu.VMEM((_NC * _L,), jnp.int32),
          pltpu.VMEM((_NC,), jnp.int32),
          pltpu.VMEM((_NC * _L, _E), jnp.float32),
          pltpu.VMEM((_NC * _L, _E), jnp.float32),
          pltpu.VMEM((_NC, _E), jnp.float32),
          pltpu.SemaphoreType.DMA,
      ],
  )
  def k(obs_hbm, myid_hbm, pos_tab, tok_tab, my_out, pos_out, tok_out,
        obs_v, myid_v, pidx_v, tidx_v, midx_v, posw_v, tokw_v, myr_v, sem):
    wid = lax.axis_index("s") * _NCORES + lax.axis_index("c")

    def chunk(ci, carry):
      base = wid * _BPW + ci * _NC
      bt = base // 128
      off = base % 128
      pltpu.sync_copy(obs_hbm.at[:, bt, :, pl.ds(off, _NC)], obs_v)
      pltpu.sync_copy(myid_hbm.at[pl.ds(base, _NC)], myid_v)

      def grp(g, c2):
        l = g // (_NC // 16)
        b0 = (g % (_NC // 16)) * 16
        bloc = b0 + lax.broadcasted_iota(jnp.int32, (16,), 0)
        r = obs_v[l, 1, pl.ds(b0, 16)]
        c = obs_v[l, 2, pl.ds(b0, 16)]
        tk = obs_v[l, 3, pl.ds(b0, 16)]
        dest = (l // 8) * (_NC * 8) + bloc * 8 + l % 8
        plsc.store_scatter(pidx_v, [dest], (r // _DS) * _GRID + (c // _DS))
        plsc.store_scatter(tidx_v, [dest], tk)
        return c2

      lax.fori_loop(0, _L * (_NC // 16), grp, 0)

      def bgrp(gb, c2):
        b0 = gb * 16
        bloc = b0 + lax.broadcasted_iota(jnp.int32, (16,), 0)
        myv = myid_v[pl.ds(b0, 16)]
        sel = jnp.zeros((16,), jnp.int32)
        found = jnp.zeros((16,), jnp.bool_)
        for l in range(_L):
          ag = obs_v[l, 0, pl.ds(b0, 16)]
          m = (ag == myv) & (ag != 0)
          sel = jnp.where(m & (~found), jnp.full((16,), l, jnp.int32), sel)
          found = found | m
        dsel = (sel // 8) * (_NC * 8) + bloc * 8 + sel % 8
        midx_v[pl.ds(b0, 16)] = plsc.load_gather(pidx_v, [dsel])
        return c2

      lax.fori_loop(0, _NC // 16, bgrp, 0)

      copies = []
      for k0 in range(0, _NC * _L, _GCHUNK):
        copies.append(pltpu.async_copy(
            pos_tab.at[pidx_v.at[pl.ds(k0, _GCHUNK)]],
            posw_v.at[pl.ds(k0, _GCHUNK)], sem))
        copies.append(pltpu.async_copy(
            tok_tab.at[tidx_v.at[pl.ds(k0, _GCHUNK)]],
            tokw_v.at[pl.ds(k0, _GCHUNK)], sem))
      copies.append(pltpu.async_copy(pos_tab.at[midx_v], myr_v, sem))
      for cp in copies:
        cp.wait()

      for j in range(_NP):
        pltpu.sync_copy(posw_v.at[pl.ds(j * _NC * 8, _NC * 8)],
                        pos_out.at[j, pl.ds(base * 8, _NC * 8)])
        pltpu.sync_copy(tokw_v.at[pl.ds(j * _NC * 8, _NC * 8)],
                        tok_out.at[j, pl.ds(base * 8, _NC * 8)])
      pltpu.sync_copy(myr_v, my_out.at[pl.ds(base, _NC)])
      return carry

    lax.fori_loop(0, _NCHUNK, chunk, 0)

  return k(obs4, my_id, pos_wide, token_wide)


def _tc_linear(myr, posr, tokr, wmy, wp, wt, bias):
  bm = 1024

  def body(my_ref, p_ref, t_ref, wmy_ref, wp_ref, wt_ref, b_ref, o_ref):
    acc = jnp.dot(my_ref[...], wmy_ref[...],
                  preferred_element_type=jnp.float32)
    for j in range(_NP):
      acc = acc + jnp.dot(p_ref[j], wp_ref[j],
                          preferred_element_type=jnp.float32)
      acc = acc + jnp.dot(t_ref[j], wt_ref[j],
                          preferred_element_type=jnp.float32)
    o_ref[...] = acc + b_ref[...]

  return pl.pallas_call(
      body,
      grid=(_B // bm,),
      in_specs=[
          pl.BlockSpec((bm, _E), lambda i: (i, 0)),
          pl.BlockSpec((_NP, bm, 128), lambda i: (0, i, 0)),
          pl.BlockSpec((_NP, bm, 128), lambda i: (0, i, 0)),
          pl.BlockSpec((_E, _OUT), lambda i: (0, 0)),
          pl.BlockSpec((_NP, 128, _OUT), lambda i: (0, 0, 0)),
          pl.BlockSpec((_NP, 128, _OUT), lambda i: (0, 0, 0)),
          pl.BlockSpec((1, _OUT), lambda i: (0, 0)),
      ],
      out_specs=pl.BlockSpec((bm, _OUT), lambda i: (i, 0)),
      out_shape=jax.ShapeDtypeStruct((_B, _OUT), jnp.float32),
  )(myr, posr, tokr, wmy, wp, wt, bias)


def kernel(comm_obs, my_id, pos_table, token_table, W, b):
  obs4 = (comm_obs.astype(jnp.int32)
          .transpose(1, 2, 0)
          .reshape(_L, 4, _B // 128, 128)
          .transpose(0, 2, 1, 3))
  myr, posr, tokr = _sc_gather(obs4, my_id.astype(jnp.int32),
                               pos_table, token_table)
  posr = posr.reshape(_NP, _B, 128)
  tokr = tokr.reshape(_NP, _B, 128)
  wmy = W[:, :_E].T
  wr = W[:, _E:].reshape(_OUT, _L, 2, _E)
  wp = wr[:, :, 0, :].reshape(_OUT, _NP, 8 * _E).transpose(1, 2, 0)
  wt = wr[:, :, 1, :].reshape(_OUT, _NP, 8 * _E).transpose(1, 2, 0)
  return _tc_linear(myr, posr, tokr, wmy, wp, wt, b.reshape(1, _OUT))
